# baseline (speedup 1.0000x reference)
import functools

import jax
import jax.numpy as jnp
from jax import lax
from jax.experimental import pallas as pl
from jax.experimental.pallas import tpu as pltpu
from jax.experimental.pallas import tpu_sc as plsc

N = 320000
D = 128
B = 1024

NC = 2
NS = 16
NW = NC * NS

C = 128
NCH = N // C
NBUF = 4
SPAN = 80
LOOP_STEPS = SPAN
PAD_CH = NW * SPAN
SEG_PER_TILE = B // NS


def _sc_segment_sum(node_embedding, ids2):
    mesh = plsc.VectorSubcoreMesh(core_axis_name="c", subcore_axis_name="s")

    @functools.partial(
        pl.kernel,
        out_type=(
            jax.ShapeDtypeStruct((NC, B, D), jnp.float32),
            jax.ShapeDtypeStruct((NW, B), jnp.float32),
        ),
        mesh=mesh,
        compiler_params=pltpu.CompilerParams(needs_layout_passes=False),
        scratch_types=[
            pltpu.VMEM((NBUF, C, D), jnp.float32),
            pltpu.VMEM((SPAN, 1, 128), jnp.int32),
            pltpu.VMEM((SEG_PER_TILE, D), jnp.float32),
            pltpu.VMEM((B,), jnp.float32),
            pltpu.VMEM_SHARED((B, D), jnp.float32),
        ] + [pltpu.SemaphoreType.DMA] * (2 * NBUF),
    )
    def seg_kernel(emb_hbm, ids_hbm, out_sum, out_cnt,
                   rows_v, idx_all, zrow_v, cnt_v, acc_sh, *sems):
        cid = lax.axis_index("c")
        sid = lax.axis_index("s")
        wid = sid * NC + cid
        sem_r = sems[0:NBUF]
        sem_s = sems[NBUF:2 * NBUF]
        base = wid * SPAN

        def fill_zrow(k, _):
            i = k // (D // 16)
            j = k % (D // 16)
            zrow_v[i, pl.ds(j * 16, 16)] = jnp.zeros((16,), jnp.float32)
            return 0
        lax.fori_loop(0, SEG_PER_TILE * (D // 16), fill_zrow, 0)

        def fill_zcnt(i, _):
            cnt_v[pl.ds(i * 16, 16)] = jnp.zeros((16,), jnp.float32)
            return 0
        lax.fori_loop(0, B // 16, fill_zcnt, 0)

        pltpu.sync_copy(zrow_v, acc_sh.at[pl.ds(sid * SEG_PER_TILE, SEG_PER_TILE)])
        pltpu.sync_copy(ids_hbm.at[pl.ds(base, SPAN)], idx_all)
        plsc.subcore_barrier()

        def issue(t, b):
            r = base + t

            @pl.when((t < SPAN) & (r < NCH))
            def _():
                pltpu.async_copy(emb_hbm.at[pl.ds(r * C, C)],
                                 rows_v.at[b], sem_r[b])

        def wait_and_process(t, b):
            r = base + t

            @pl.when((t < SPAN) & (r < NCH))
            def _():
                pltpu.make_async_copy(emb_hbm.at[pl.ds(r * C, C)],
                                      rows_v.at[b], sem_r[b]).wait()
                pltpu.async_copy(rows_v.at[b], acc_sh.at[idx_all.at[t, 0]],
                                 sem_s[b], add=True)
                for g in range(128 // 16):
                    idx16 = idx_all[t, 0, pl.ds(g * 16, 16)]
                    plsc.addupdate_scatter(cnt_v, [idx16],
                                           jnp.ones((16,), jnp.float32))

        def drain_scatter(t, b):
            r = base + t

            @pl.when((t >= 0) & (t < SPAN) & (r < NCH))
            def _():
                pltpu.make_async_copy(rows_v.at[b],
                                      acc_sh.at[idx_all.at[t, 0]],
                                      sem_s[b]).wait()

        issue(0, 0)
        issue(1, 1)

        def body(g, _):
            for bb in range(NBUF):
                t = NBUF * g + bb
                wait_and_process(t, bb)
                drain_scatter(t - 2, (bb - 2) % NBUF)
                issue(t + 2, (bb + 2) % NBUF)
            return 0
        lax.fori_loop(0, LOOP_STEPS // NBUF, body, 0)

        drain_scatter(LOOP_STEPS - 2, (LOOP_STEPS - 2) % NBUF)
        drain_scatter(LOOP_STEPS - 1, (LOOP_STEPS - 1) % NBUF)

        plsc.subcore_barrier()

        s0 = sid * SEG_PER_TILE
        pltpu.sync_copy(acc_sh.at[pl.ds(s0, SEG_PER_TILE)],
                        out_sum.at[cid, pl.ds(s0, SEG_PER_TILE)])
        pltpu.sync_copy(cnt_v, out_cnt.at[wid])

    return seg_kernel(node_embedding, ids2)


def _tc_readout(psum, pcnt, W1, b1, W2, b2):
    def body(ps_ref, pc_ref, w1_ref, b1_ref, w2_ref, b2_ref, out_ref):
        sums = ps_ref[0] + ps_ref[1]
        counts = jnp.maximum(jnp.sum(pc_ref[...], axis=0), 1.0)
        g = sums / counts[:, None]
        h = jnp.dot(g, w1_ref[...], preferred_element_type=jnp.float32)
        h = jnp.maximum(h + b1_ref[0, :], 0.0)
        o = jnp.dot(h, w2_ref[...], preferred_element_type=jnp.float32)
        o = o + b2_ref[0, 0]
        out_ref[...] = 1.0 / (1.0 + jnp.exp(-o))

    return pl.pallas_call(
        body,
        out_shape=jax.ShapeDtypeStruct((B, 1), jnp.float32),
    )(psum, pcnt, W1, b1, W2, b2)


def kernel(node_embedding, segment_ids, W1, b1, W2, b2):
    ids2 = segment_ids.astype(jnp.int32).reshape(N // 128, 128)
    ids2 = jnp.concatenate(
        [ids2, jnp.zeros((PAD_CH - N // 128, 128), jnp.int32)])
    ids2 = ids2.reshape(PAD_CH, 1, 128)
    psum, pcnt = _sc_segment_sum(node_embedding, ids2)
    out = _tc_readout(psum, pcnt, W1, b1.reshape(1, D), W2, b2.reshape(1, 1))
    return out[:, 0]


# speedup vs baseline: 1.1335x; 1.1335x over previous
import functools

import jax
import jax.numpy as jnp
from jax import lax
from jax.experimental import pallas as pl
from jax.experimental.pallas import tpu as pltpu
from jax.experimental.pallas import tpu_sc as plsc

N = 320000
D = 128
B = 1024

NC = 2
NS = 16
NW = NC * NS

C = 128
NCH = N // C
NBUF = 6
CHUNKS_PER_WORKER = -(-(NCH) // NW)
LOOP_STEPS = -(-CHUNKS_PER_WORKER // NBUF) * NBUF
IDROWS_PER_CHUNK = C // 128
SEG_PER_TILE = B // NS


def _sc_segment_sum(node_embedding, ids2):
    mesh = plsc.VectorSubcoreMesh(core_axis_name="c", subcore_axis_name="s")

    @functools.partial(
        pl.kernel,
        out_type=(
            jax.ShapeDtypeStruct((NC, B, D), jnp.float32),
            jax.ShapeDtypeStruct((NW, B), jnp.float32),
        ),
        mesh=mesh,
        compiler_params=pltpu.CompilerParams(needs_layout_passes=False),
        scratch_types=[
            pltpu.VMEM((NBUF, C, D), jnp.float32),
            pltpu.VMEM((NBUF, IDROWS_PER_CHUNK, 128), jnp.int32),
            pltpu.VMEM((SEG_PER_TILE, D), jnp.float32),
            pltpu.VMEM((B,), jnp.float32),
            pltpu.VMEM_SHARED((B, D), jnp.float32),
        ] + [pltpu.SemaphoreType.DMA] * (3 * NBUF),
    )
    def seg_kernel(emb_hbm, ids_hbm, out_sum, out_cnt,
                   rows_v, idx_v, zrow_v, cnt_v, acc_sh, *sems):
        cid = lax.axis_index("c")
        sid = lax.axis_index("s")
        wid = sid * NC + cid
        sem_r = sems[0:NBUF]
        sem_i = sems[NBUF:2 * NBUF]
        sem_s = sems[2 * NBUF:3 * NBUF]

        def fill_zrow(k, _):
            i = k // (D // 16)
            j = k % (D // 16)
            zrow_v[i, pl.ds(j * 16, 16)] = jnp.zeros((16,), jnp.float32)
            return 0
        lax.fori_loop(0, SEG_PER_TILE * (D // 16), fill_zrow, 0)

        def fill_zcnt(i, _):
            cnt_v[pl.ds(i * 16, 16)] = jnp.zeros((16,), jnp.float32)
            return 0
        lax.fori_loop(0, B // 16, fill_zcnt, 0)

        pltpu.sync_copy(zrow_v, acc_sh.at[pl.ds(sid * SEG_PER_TILE, SEG_PER_TILE)])
        plsc.subcore_barrier()

        def issue(t, b):
            r = t * NW + wid

            @pl.when(r < NCH)
            def _():
                pltpu.async_copy(
                    ids_hbm.at[pl.ds(r * IDROWS_PER_CHUNK, IDROWS_PER_CHUNK)],
                    idx_v.at[b], sem_i[b])
                pltpu.async_copy(emb_hbm.at[pl.ds(r * C, C)],
                                 rows_v.at[b], sem_r[b])

        def wait_and_process(t, b):
            r = t * NW + wid

            @pl.when(r < NCH)
            def _():
                pltpu.make_async_copy(
                    ids_hbm.at[pl.ds(r * IDROWS_PER_CHUNK, IDROWS_PER_CHUNK)],
                    idx_v.at[b], sem_i[b]).wait()
                pltpu.make_async_copy(emb_hbm.at[pl.ds(r * C, C)],
                                      rows_v.at[b], sem_r[b]).wait()
                for j in range(IDROWS_PER_CHUNK):
                    pltpu.async_copy(rows_v.at[b, pl.ds(j * 128, 128)],
                                     acc_sh.at[idx_v.at[b, j]], sem_s[b],
                                     add=True)
                    for g in range(128 // 16):
                        idx16 = idx_v[b, j, pl.ds(g * 16, 16)]
                        plsc.addupdate_scatter(cnt_v, [idx16],
                                               jnp.ones((16,), jnp.float32))

        def drain_scatter(t, b):
            r = t * NW + wid

            @pl.when((r >= 0) & (r < NCH))
            def _():
                for j in range(IDROWS_PER_CHUNK):
                    pltpu.make_async_copy(
                        rows_v.at[b, pl.ds(j * 128, 128)],
                        acc_sh.at[idx_v.at[b, j]], sem_s[b]).wait()

        issue(0, 0)
        issue(1, 1)
        issue(2, 2)
        issue(3, 3)

        def body(g, _):
            for bb in range(NBUF):
                t = NBUF * g + bb
                wait_and_process(t, bb)
                drain_scatter(t - 2, (bb - 2) % NBUF)
                issue(t + 4, (bb + 4) % NBUF)
            return 0
        lax.fori_loop(0, LOOP_STEPS // NBUF, body, 0)

        drain_scatter(LOOP_STEPS - 2, (LOOP_STEPS - 2) % NBUF)
        drain_scatter(LOOP_STEPS - 1, (LOOP_STEPS - 1) % NBUF)

        plsc.subcore_barrier()

        s0 = sid * SEG_PER_TILE
        pltpu.sync_copy(acc_sh.at[pl.ds(s0, SEG_PER_TILE)],
                        out_sum.at[cid, pl.ds(s0, SEG_PER_TILE)])
        pltpu.sync_copy(cnt_v, out_cnt.at[wid])

    return seg_kernel(node_embedding, ids2)


def _tc_readout(psum, pcnt, W1, b1, W2, b2):
    def body(ps_ref, pc_ref, w1_ref, b1_ref, w2_ref, b2_ref, out_ref):
        sums = ps_ref[0] + ps_ref[1]
        counts = jnp.maximum(jnp.sum(pc_ref[...], axis=0), 1.0)
        g = sums / counts[:, None]
        h = jnp.dot(g, w1_ref[...], preferred_element_type=jnp.float32)
        h = jnp.maximum(h + b1_ref[0, :], 0.0)
        o = jnp.dot(h, w2_ref[...], preferred_element_type=jnp.float32)
        o = o + b2_ref[0, 0]
        out_ref[...] = 1.0 / (1.0 + jnp.exp(-o))

    return pl.pallas_call(
        body,
        out_shape=jax.ShapeDtypeStruct((B, 1), jnp.float32),
    )(psum, pcnt, W1, b1, W2, b2)


def kernel(node_embedding, segment_ids, W1, b1, W2, b2):
    ids2 = segment_ids.astype(jnp.int32).reshape(N // 128, 128)
    psum, pcnt = _sc_segment_sum(node_embedding, ids2)
    out = _tc_readout(psum, pcnt, W1, b1.reshape(1, D), W2, b2.reshape(1, 1))
    return out[:, 0]
